# baseline (speedup 1.0000x reference)
import functools

import jax
import jax.numpy as jnp
from jax.experimental import pallas as pl

H = 64
BLK = 1000


def _row_call(body, rows, row_ins, full_ins, out_widths, blk=BLK):
    grid = (rows // blk,)
    in_specs = [pl.BlockSpec((blk, a.shape[1]), lambda i: (i, 0)) for a in row_ins]
    in_specs += [pl.BlockSpec(a.shape, lambda i: (0, 0)) for a in full_ins]
    out_specs = [pl.BlockSpec((blk, w), lambda i: (i, 0)) for w in out_widths]
    out_shape = [jax.ShapeDtypeStruct((rows, w), jnp.float32) for w in out_widths]
    f = pl.pallas_call(
        body,
        grid=grid,
        in_specs=in_specs,
        out_specs=out_specs,
        out_shape=out_shape,
    )
    return f(*row_ins, *full_ins)


def _dot(a, b):
    return jax.lax.dot(a, b, precision=jax.lax.Precision.HIGHEST)


def _ln(v, g, b):
    mu = jnp.mean(v, axis=-1, keepdims=True)
    var = jnp.mean((v - mu) ** 2, axis=-1, keepdims=True)
    return (v - mu) / jnp.sqrt(var + 1e-5) * g + b


def _silu(v):
    return v * jax.nn.sigmoid(v)


def _mlp_blk(v, w, b, g, bb):
    return _silu(_ln(_dot(v, w) + b, g, bb))


def _emb_body(a_ref, w_ref, b_ref, g_ref, bb_ref, o_ref):
    o_ref[...] = _mlp_blk(a_ref[...], w_ref[...], b_ref[...], g_ref[...], bb_ref[...])


def _rbf(x, vmin, vmax, bins):
    step = (vmax - vmin) / (bins - 1)
    idx = jax.lax.broadcasted_iota(jnp.int32, (1, bins), 1).astype(jnp.float32)
    centers = vmin + idx * step
    gamma = 1.0 / step**2
    return jnp.exp(-gamma * (x - centers) ** 2)


def _edge_emb_body(r_ref, w1, b1, g1, bb1, w2, b2, g2, bb2, o_ref):
    r = r_ref[...]
    d = jnp.sqrt(jnp.sum(r * r, axis=1, keepdims=True) + 1e-12)
    y = _rbf(d, 0.0, 8.0, 80)
    y = _mlp_blk(y, w1[...], b1[...], g1[...], bb1[...])
    o_ref[...] = _mlp_blk(y, w2[...], b2[...], g2[...], bb2[...])


def _angle_emb_body(c_ref, w1, b1, g1, bb1, w2, b2, g2, bb2, o_ref):
    z = _rbf(c_ref[...], -1.0, 1.0, 40)
    z = _mlp_blk(z, w1[...], b1[...], g1[...], bb1[...])
    o_ref[...] = _mlp_blk(z, w2[...], b2[...], g2[...], bb2[...])


def _pre_body(x_ref, wa, ba, wb, bb_, wd, bd, wc, bc, src_ref, dst_ref, cup_ref):
    x = x_ref[...]
    a = _dot(x, wa[...]) + ba[...]
    bh = _dot(x, wb[...]) + bb_[...]
    src_ref[...] = jnp.concatenate([a, bh], axis=-1)
    dst_ref[...] = _dot(x, wd[...]) + bd[...]
    cup_ref[...] = _dot(x, wc[...]) + bc[...]


def _mid_body(ga_ref, gd_ref, y_ref, we, be, g, bb_, v_ref, ynew_ref):
    ga = ga_ref[...]
    y = y_ref[...]
    m = ga[:, :H] + gd_ref[...] + (_dot(y, we[...]) + be[...])
    sig = jax.nn.sigmoid(m)
    v_ref[...] = jnp.concatenate([sig * ga[:, H:], sig], axis=-1)
    ynew_ref[...] = y + _silu(_ln(m, g[...], bb_[...]))


def _post_body(s_ref, cup_ref, x_ref, g, bb_, xnew_ref):
    s = s_ref[...]
    h = s[:, :H] / (s[:, H:] + 1e-6)
    xnew_ref[...] = x_ref[...] + _silu(_ln(cup_ref[...] + h, g[...], bb_[...]))


def _final_body(x_ref, w_ref, b_ref, o_ref):
    n = x_ref.shape[0]
    h = jnp.sum(x_ref[...], axis=0, keepdims=True) * (1.0 / n)
    o_ref[...] = (_dot(h, w_ref[...]) + b_ref[...]) * float(n)


def _p2(p):
    return p["W"], p["b"].reshape(1, -1)


def _ln2(p):
    return p["g"].reshape(1, -1), p["b"].reshape(1, -1)


def _gather(table, idx):
    return jnp.take(table, idx, axis=0)


def _segsum(vals, idx, n_seg):
    return jax.ops.segment_sum(vals, idx, num_segments=n_seg)


def _conv(p, src, dst, n_seg, x, y):
    n_x = x.shape[0]
    n_e = y.shape[0]
    srcbuf, dstbuf, cup = _row_call(
        _pre_body, n_x, [x],
        [*_p2(p["src_gate"]), *_p2(p["dst_update"]), *_p2(p["dst_gate"]),
         *_p2(p["src_update"])],
        [2 * H, H, H])
    ga = _gather(srcbuf, src)
    gd = _gather(dstbuf, dst)
    v, ynew = _row_call(
        _mid_body, n_e, [ga, gd, y],
        [*_p2(p["edge_gate"]), *_ln2(p["ln_edges"])],
        [2 * H, H])
    s = _segsum(v, dst, n_seg)
    (xnew,) = _row_call(
        _post_body, n_x, [s, cup, x], [*_ln2(p["ln_nodes"])], [H])
    return xnew, ynew


def kernel(atom_features, edge_index, r, lg_edge_index, angle_cos, params):
    src, dst = edge_index[0], edge_index[1]
    lsrc, ldst = lg_edge_index[0], lg_edge_index[1]
    n_nodes = atom_features.shape[0]
    n_edges = r.shape[0]

    pa = params["atom_emb"]
    (x,) = _row_call(
        _emb_body, n_nodes, [atom_features],
        [*_p2(pa["lin"]), *_ln2(pa["ln"])], [H])

    pe = params["edge_emb"]
    (y,) = _row_call(
        _edge_emb_body, n_edges, [r],
        [*_p2(pe[0]["lin"]), *_ln2(pe[0]["ln"]),
         *_p2(pe[1]["lin"]), *_ln2(pe[1]["ln"])], [H])

    pz = params["angle_emb"]
    (z,) = _row_call(
        _angle_emb_body, angle_cos.shape[0], [angle_cos.reshape(-1, 1)],
        [*_p2(pz[0]["lin"]), *_ln2(pz[0]["ln"]),
         *_p2(pz[1]["lin"]), *_ln2(pz[1]["ln"])], [H])

    for layer in params["alignn"]:
        x, m = _conv(layer["node"], src, dst, n_nodes, x, y)
        y, z = _conv(layer["edge"], lsrc, ldst, n_edges, m, z)

    for p in params["gcn"]:
        x, y = _conv(p, src, dst, n_nodes, x, y)

    pf = params["fc"]
    out = pl.pallas_call(
        _final_body,
        in_specs=[pl.BlockSpec((n_nodes, H), lambda: (0, 0)),
                  pl.BlockSpec((H, 1), lambda: (0, 0)),
                  pl.BlockSpec((1, 1), lambda: (0, 0))],
        out_specs=pl.BlockSpec((1, 1), lambda: (0, 0)),
        out_shape=jax.ShapeDtypeStruct((1, 1), jnp.float32),
    )(x, pf["W"], pf["b"].reshape(1, 1))
    return out.reshape(1)


# speedup vs baseline: 1.0406x; 1.0406x over previous
import functools

import jax
import jax.numpy as jnp
from jax import lax
from jax.experimental import pallas as pl
from jax.experimental.pallas import tpu as pltpu
from jax.experimental.pallas import tpu_sc as plsc

H = 64
BLK = 1000
_NC, _NS = 2, 16
_NW = _NC * _NS


def _row_call(body, rows, row_ins, full_ins, out_widths, blk=BLK):
    grid = (rows // blk,)
    in_specs = [pl.BlockSpec((blk, a.shape[1]), lambda i: (i, 0)) for a in row_ins]
    in_specs += [pl.BlockSpec(a.shape, lambda i: (0, 0)) for a in full_ins]
    out_specs = [pl.BlockSpec((blk, w), lambda i: (i, 0)) for w in out_widths]
    out_shape = [jax.ShapeDtypeStruct((rows, w), jnp.float32) for w in out_widths]
    f = pl.pallas_call(
        body,
        grid=grid,
        in_specs=in_specs,
        out_specs=out_specs,
        out_shape=out_shape,
    )
    return f(*row_ins, *full_ins)


def _dot(a, b):
    return jax.lax.dot(a, b, precision=jax.lax.Precision.HIGHEST)


def _ln(v, g, b):
    mu = jnp.mean(v, axis=-1, keepdims=True)
    var = jnp.mean((v - mu) ** 2, axis=-1, keepdims=True)
    return (v - mu) / jnp.sqrt(var + 1e-5) * g + b


def _silu(v):
    return v * jax.nn.sigmoid(v)


def _mlp_blk(v, w, b, g, bb):
    return _silu(_ln(_dot(v, w) + b, g, bb))


def _emb_body(a_ref, w_ref, b_ref, g_ref, bb_ref, o_ref):
    o_ref[...] = _mlp_blk(a_ref[...], w_ref[...], b_ref[...], g_ref[...], bb_ref[...])


def _rbf(x, vmin, vmax, bins):
    step = (vmax - vmin) / (bins - 1)
    idx = jax.lax.broadcasted_iota(jnp.int32, (1, bins), 1).astype(jnp.float32)
    centers = vmin + idx * step
    gamma = 1.0 / step**2
    return jnp.exp(-gamma * (x - centers) ** 2)


def _edge_emb_body(r_ref, w1, b1, g1, bb1, w2, b2, g2, bb2, o_ref):
    r = r_ref[...]
    d = jnp.sqrt(jnp.sum(r * r, axis=1, keepdims=True) + 1e-12)
    y = _rbf(d, 0.0, 8.0, 80)
    y = _mlp_blk(y, w1[...], b1[...], g1[...], bb1[...])
    o_ref[...] = _mlp_blk(y, w2[...], b2[...], g2[...], bb2[...])


def _angle_emb_body(c_ref, w1, b1, g1, bb1, w2, b2, g2, bb2, o_ref):
    z = _rbf(c_ref[...], -1.0, 1.0, 40)
    z = _mlp_blk(z, w1[...], b1[...], g1[...], bb1[...])
    o_ref[...] = _mlp_blk(z, w2[...], b2[...], g2[...], bb2[...])


def _pre_body(x_ref, wa, ba, wb, bb_, wd, bd, wc, bc, src_ref, dst_ref, cup_ref):
    x = x_ref[...]
    a = _dot(x, wa[...]) + ba[...]
    bh = _dot(x, wb[...]) + bb_[...]
    src_ref[...] = jnp.concatenate([a, bh], axis=-1)
    d = _dot(x, wd[...]) + bd[...]
    dst_ref[...] = jnp.concatenate([d, jnp.zeros_like(d)], axis=-1)
    cup_ref[...] = _dot(x, wc[...]) + bc[...]


def _mid_body(ga_ref, gd_ref, y_ref, we, be, g, bb_, v0_ref, v1_ref, ynew_ref):
    ga = ga_ref[...]
    y = y_ref[...]
    m = ga[:, :H] + gd_ref[...][:, :H] + (_dot(y, we[...]) + be[...])
    sig = jax.nn.sigmoid(m)
    v0_ref[...] = sig * ga[:, H:]
    v1_ref[...] = sig
    ynew_ref[...] = y + _silu(_ln(m, g[...], bb_[...]))


def _post_body2(sh_ref, ss_ref, cup_ref, x_ref, g, bb_, xnew_ref):
    h = sh_ref[...] / (ss_ref[...] + 1e-6)
    xnew_ref[...] = x_ref[...] + _silu(_ln(cup_ref[...] + h, g[...], bb_[...]))


def _post_body4(sh0_ref, sh1_ref, ss0_ref, ss1_ref, cup_ref, x_ref, g, bb_,
                xnew_ref):
    h = (sh0_ref[...] + sh1_ref[...]) / (ss0_ref[...] + ss1_ref[...] + 1e-6)
    xnew_ref[...] = x_ref[...] + _silu(_ln(cup_ref[...] + h, g[...], bb_[...]))


def _final_body(x_ref, w_ref, b_ref, o_ref):
    n = x_ref.shape[0]
    h = jnp.sum(x_ref[...], axis=0, keepdims=True) * (1.0 / n)
    o_ref[...] = (_dot(h, w_ref[...]) + b_ref[...]) * float(n)


def _p2(p):
    return p["W"], p["b"].reshape(1, -1)


def _ln2(p):
    return p["g"].reshape(1, -1), p["b"].reshape(1, -1)


def _sc_gather_call(table, idx, chunk):
    B = idx.shape[0]
    D = table.shape[1]
    per_w = B // _NW
    n_chunks = per_w // chunk
    mesh = plsc.VectorSubcoreMesh(core_axis_name="c", subcore_axis_name="s")

    @functools.partial(
        pl.kernel, mesh=mesh,
        out_type=jax.ShapeDtypeStruct((B, D), jnp.float32),
        scratch_types=[
            pltpu.VMEM((chunk,), jnp.int32),
            pltpu.VMEM((chunk, D), jnp.float32),
            pltpu.SemaphoreType.DMA,
        ],
    )
    def k(table_hbm, idx_hbm, out_hbm, idx_v, rows_v, sem):
        wid = lax.axis_index("s") * _NC + lax.axis_index("c")
        base = wid * per_w

        def body(j, carry):
            off = base + j * chunk
            pltpu.sync_copy(idx_hbm.at[pl.ds(off, chunk)], idx_v)
            pltpu.async_copy(table_hbm.at[idx_v], rows_v, sem).wait()
            pltpu.sync_copy(rows_v, out_hbm.at[pl.ds(off, chunk)])
            return carry

        lax.fori_loop(0, n_chunks, body, 0)

    return k(table, idx)


def _gather(table, idx):
    B = idx.shape[0]
    chunk = 400 if B % (400 * _NW) == 0 else 384
    unit = chunk * _NW
    Bp = ((B + unit - 1) // unit) * unit
    if Bp != B:
        pad = jnp.arange(Bp - B, dtype=jnp.int32) % table.shape[0]
        idx = jnp.concatenate([idx, pad])
    out = _sc_gather_call(table, idx, chunk)
    return out[:B] if Bp != B else out


def _sc_scatter_small(v0, v1, dst, n_seg, zeros):
    E, D = v0.shape
    per_w = E // _NW
    chunk = 400
    n_chunks = per_w // chunk
    mesh = plsc.VectorSubcoreMesh(core_axis_name="c", subcore_axis_name="s")

    @functools.partial(
        pl.kernel, mesh=mesh,
        out_type=jax.ShapeDtypeStruct((_NC, 2, n_seg, D), jnp.float32),
        scratch_types=[
            pltpu.VMEM((chunk,), jnp.int32),
            pltpu.VMEM((chunk, D), jnp.float32),
            pltpu.VMEM_SHARED((n_seg, D), jnp.float32),
            pltpu.SemaphoreType.DMA,
        ],
    )
    def k(v0_hbm, v1_hbm, dst_hbm, zeros_hbm, out_hbm, idx_v, rows_v, acc, sem):
        cid = lax.axis_index("c")
        sid = lax.axis_index("s")
        wid = sid * _NC + cid
        base = wid * per_w

        for phase, vals_hbm in ((0, v0_hbm), (1, v1_hbm)):
            @pl.when(sid == 0)
            def _zero():
                pltpu.sync_copy(zeros_hbm, acc)

            plsc.subcore_barrier()

            def body(j, carry):
                off = base + j * chunk
                pltpu.sync_copy(dst_hbm.at[pl.ds(off, chunk)], idx_v)
                pltpu.sync_copy(vals_hbm.at[pl.ds(off, chunk)], rows_v)
                pltpu.sync_copy(rows_v, acc.at[idx_v], add=True)
                return carry

            lax.fori_loop(0, n_chunks, body, 0)
            plsc.subcore_barrier()

            @pl.when(sid == 0)
            def _flush():
                pltpu.sync_copy(acc, out_hbm.at[cid].at[phase])

            plsc.subcore_barrier()

    return k(v0, v1, dst, zeros)


def _segsum_pair(v0, v1, idx, n_seg):
    if False and n_seg * H * 4 <= 4 * 2**20 and v0.shape[0] % (400 * _NW) == 0:
        zeros = jnp.zeros((n_seg, H), jnp.float32)
        parts = _sc_scatter_small(v0, v1, idx, n_seg, zeros)
        return [parts[0, 0], parts[1, 0], parts[0, 1], parts[1, 1]], _post_body4
    sh = jax.ops.segment_sum(v0, idx, num_segments=n_seg)
    ss = jax.ops.segment_sum(v1, idx, num_segments=n_seg)
    return [sh, ss], _post_body2


def _conv(p, src, dst, n_seg, x, y):
    n_x = x.shape[0]
    n_e = y.shape[0]
    srcbuf, dstbuf, cup = _row_call(
        _pre_body, n_x, [x],
        [*_p2(p["src_gate"]), *_p2(p["dst_update"]), *_p2(p["dst_gate"]),
         *_p2(p["src_update"])],
        [2 * H, 2 * H, H])
    ga = _gather(srcbuf, src)
    gd = _gather(dstbuf, dst)
    v0, v1, ynew = _row_call(
        _mid_body, n_e, [ga, gd, y],
        [*_p2(p["edge_gate"]), *_ln2(p["ln_edges"])],
        [H, H, H])
    s_rows, post = _segsum_pair(v0, v1, dst, n_seg)
    (xnew,) = _row_call(
        post, n_x, [*s_rows, cup, x], [*_ln2(p["ln_nodes"])], [H])
    return xnew, ynew


def kernel(atom_features, edge_index, r, lg_edge_index, angle_cos, params):
    src, dst = edge_index[0], edge_index[1]
    lsrc, ldst = lg_edge_index[0], lg_edge_index[1]
    n_nodes = atom_features.shape[0]
    n_edges = r.shape[0]

    pa = params["atom_emb"]
    (x,) = _row_call(
        _emb_body, n_nodes, [atom_features],
        [*_p2(pa["lin"]), *_ln2(pa["ln"])], [H])

    pe = params["edge_emb"]
    (y,) = _row_call(
        _edge_emb_body, n_edges, [r],
        [*_p2(pe[0]["lin"]), *_ln2(pe[0]["ln"]),
         *_p2(pe[1]["lin"]), *_ln2(pe[1]["ln"])], [H])

    pz = params["angle_emb"]
    (z,) = _row_call(
        _angle_emb_body, angle_cos.shape[0], [angle_cos.reshape(-1, 1)],
        [*_p2(pz[0]["lin"]), *_ln2(pz[0]["ln"]),
         *_p2(pz[1]["lin"]), *_ln2(pz[1]["ln"])], [H])

    for layer in params["alignn"]:
        x, m = _conv(layer["node"], src, dst, n_nodes, x, y)
        y, z = _conv(layer["edge"], lsrc, ldst, n_edges, m, z)

    for p in params["gcn"]:
        x, y = _conv(p, src, dst, n_nodes, x, y)

    pf = params["fc"]
    out = pl.pallas_call(
        _final_body,
        in_specs=[pl.BlockSpec((n_nodes, H), lambda: (0, 0)),
                  pl.BlockSpec((H, 1), lambda: (0, 0)),
                  pl.BlockSpec((1, 1), lambda: (0, 0))],
        out_specs=pl.BlockSpec((1, 1), lambda: (0, 0)),
        out_shape=jax.ShapeDtypeStruct((1, 1), jnp.float32),
    )(x, pf["W"], pf["b"].reshape(1, 1))
    return out.reshape(1)


# speedup vs baseline: 1.1475x; 1.1027x over previous
import functools

import jax
import jax.numpy as jnp
from jax import lax
from jax.experimental import pallas as pl
from jax.experimental.pallas import tpu as pltpu
from jax.experimental.pallas import tpu_sc as plsc

H = 64
BLK = 1000
_NC, _NS = 2, 16
_NW = _NC * _NS


def _row_call(body, rows, row_ins, full_ins, out_widths, blk=BLK):
    grid = (rows // blk,)
    in_specs = [pl.BlockSpec((blk, a.shape[1]), lambda i: (i, 0)) for a in row_ins]
    in_specs += [pl.BlockSpec(a.shape, lambda i: (0, 0)) for a in full_ins]
    out_specs = [pl.BlockSpec((blk, w), lambda i: (i, 0)) for w in out_widths]
    out_shape = [jax.ShapeDtypeStruct((rows, w), jnp.float32) for w in out_widths]
    f = pl.pallas_call(
        body,
        grid=grid,
        in_specs=in_specs,
        out_specs=out_specs,
        out_shape=out_shape,
    )
    return f(*row_ins, *full_ins)


def _dot(a, b):
    return jax.lax.dot(a, b, precision=jax.lax.Precision.HIGHEST)


def _ln(v, g, b):
    mu = jnp.mean(v, axis=-1, keepdims=True)
    var = jnp.mean((v - mu) ** 2, axis=-1, keepdims=True)
    return (v - mu) / jnp.sqrt(var + 1e-5) * g + b


def _silu(v):
    return v * jax.nn.sigmoid(v)


def _mlp_blk(v, w, b, g, bb):
    return _silu(_ln(_dot(v, w) + b, g, bb))


def _emb_body(a_ref, w_ref, b_ref, g_ref, bb_ref, o_ref):
    o_ref[...] = _mlp_blk(a_ref[...], w_ref[...], b_ref[...], g_ref[...], bb_ref[...])


def _rbf(x, vmin, vmax, bins):
    step = (vmax - vmin) / (bins - 1)
    idx = jax.lax.broadcasted_iota(jnp.int32, (1, bins), 1).astype(jnp.float32)
    centers = vmin + idx * step
    gamma = 1.0 / step**2
    return jnp.exp(-gamma * (x - centers) ** 2)


def _edge_emb_body(r_ref, w1, b1, g1, bb1, w2, b2, g2, bb2, o_ref):
    r = r_ref[...]
    d = jnp.sqrt(jnp.sum(r * r, axis=1, keepdims=True) + 1e-12)
    y = _rbf(d, 0.0, 8.0, 80)
    y = _mlp_blk(y, w1[...], b1[...], g1[...], bb1[...])
    o_ref[...] = _mlp_blk(y, w2[...], b2[...], g2[...], bb2[...])


def _angle_emb_body(c_ref, w1, b1, g1, bb1, w2, b2, g2, bb2, o_ref):
    z = _rbf(c_ref[...], -1.0, 1.0, 40)
    z = _mlp_blk(z, w1[...], b1[...], g1[...], bb1[...])
    o_ref[...] = _mlp_blk(z, w2[...], b2[...], g2[...], bb2[...])


def _pre_body(x_ref, wa, ba, wb, bb_, wd, bd, wc, bc, src_ref, dst_ref, cup_ref):
    x = x_ref[...]
    a = _dot(x, wa[...]) + ba[...]
    bh = _dot(x, wb[...]) + bb_[...]
    src_ref[...] = jnp.concatenate([a, bh], axis=-1)
    d = _dot(x, wd[...]) + bd[...]
    dst_ref[...] = jnp.concatenate([d, jnp.zeros_like(d)], axis=-1)
    cup_ref[...] = _dot(x, wc[...]) + bc[...]


def _mid_body(ga_ref, gd_ref, y_ref, we, be, g, bb_, v_ref, ynew_ref):
    ga = ga_ref[...]
    y = y_ref[...]
    m = ga[:, :H] + gd_ref[...][:, :H] + (_dot(y, we[...]) + be[...])
    sig = jax.nn.sigmoid(m)
    v_ref[...] = jnp.concatenate([sig * ga[:, H:], sig], axis=-1)
    ynew_ref[...] = y + _silu(_ln(m, g[...], bb_[...]))


def _post_body(s_ref, cup_ref, x_ref, g, bb_, xnew_ref):
    sv = s_ref[...]
    h = sv[:, :H] / (sv[:, H:] + 1e-6)
    xnew_ref[...] = x_ref[...] + _silu(_ln(cup_ref[...] + h, g[...], bb_[...]))


def _final_body(x_ref, w_ref, b_ref, o_ref):
    n = x_ref.shape[0]
    h = jnp.sum(x_ref[...], axis=0, keepdims=True) * (1.0 / n)
    o_ref[...] = (_dot(h, w_ref[...]) + b_ref[...]) * float(n)


def _p2(p):
    return p["W"], p["b"].reshape(1, -1)


def _ln2(p):
    return p["g"].reshape(1, -1), p["b"].reshape(1, -1)


def _sc_gather_call(table, idx, chunk):
    B = idx.shape[0]
    D = table.shape[1]
    per_w = B // _NW
    n_chunks = per_w // chunk
    mesh = plsc.VectorSubcoreMesh(core_axis_name="c", subcore_axis_name="s")

    @functools.partial(
        pl.kernel, mesh=mesh,
        out_type=jax.ShapeDtypeStruct((B, D), jnp.float32),
        scratch_types=[
            pltpu.VMEM((chunk,), jnp.int32),
            pltpu.VMEM((chunk, D), jnp.float32),
            pltpu.SemaphoreType.DMA,
        ],
    )
    def k(table_hbm, idx_hbm, out_hbm, idx_v, rows_v, sem):
        wid = lax.axis_index("s") * _NC + lax.axis_index("c")
        base = wid * per_w

        def body(j, carry):
            off = base + j * chunk
            pltpu.sync_copy(idx_hbm.at[pl.ds(off, chunk)], idx_v)
            pltpu.async_copy(table_hbm.at[idx_v], rows_v, sem).wait()
            pltpu.sync_copy(rows_v, out_hbm.at[pl.ds(off, chunk)])
            return carry

        lax.fori_loop(0, n_chunks, body, 0)

    return k(table, idx)


def _gather(table, idx):
    B = idx.shape[0]
    chunk = 400 if B % (400 * _NW) == 0 else 384
    unit = chunk * _NW
    Bp = ((B + unit - 1) // unit) * unit
    if Bp != B:
        pad = jnp.arange(Bp - B, dtype=jnp.int32) % table.shape[0]
        idx = jnp.concatenate([idx, pad])
    out = _sc_gather_call(table, idx, chunk)
    return out[:B] if Bp != B else out


def _sc_scatter_small(vals, dst, n_seg, zeros):
    E, D = vals.shape
    half = n_seg // _NC
    n_dump = 4096
    acc_rows = half + n_dump + 8
    per_t = E // _NS
    chunk = 400
    n_chunks = per_t // chunk
    nvec = chunk // 16
    mesh = plsc.VectorSubcoreMesh(core_axis_name="c", subcore_axis_name="s")

    @functools.partial(
        pl.kernel, mesh=mesh,
        out_type=jax.ShapeDtypeStruct((_NC, half, D), jnp.float32),
        scratch_types=[
            pltpu.VMEM((chunk,), jnp.int32),
            pltpu.VMEM((chunk,), jnp.int32),
            pltpu.VMEM((chunk, D), jnp.float32),
            pltpu.VMEM_SHARED((acc_rows, D), jnp.float32),
            pltpu.SemaphoreType.DMA,
        ],
    )
    def k(vals_hbm, dst_hbm, zeros_hbm, out_hbm, idx_v, lidx_v, rows_v, acc,
          sem):
        cid = lax.axis_index("c")
        sid = lax.axis_index("s")
        base = sid * per_t
        lo = cid * half

        @pl.when(sid == 0)
        def _zero():
            pltpu.sync_copy(zeros_hbm, acc)

        plsc.subcore_barrier()

        def body(j, carry):
            off = base + j * chunk
            pltpu.sync_copy(dst_hbm.at[pl.ds(off, chunk)], idx_v)
            pltpu.sync_copy(vals_hbm.at[pl.ds(off, chunk)], rows_v)
            for i in range(nvec):
                iv = idx_v[pl.ds(i * 16, 16)]
                local = iv - lo
                oor = (local < 0) | (local >= half)
                dump = half + (iv & (n_dump - 1))
                lidx_v[pl.ds(i * 16, 16)] = jnp.where(oor, dump, local)
            pltpu.sync_copy(rows_v, acc.at[lidx_v], add=True)
            return carry

        lax.fori_loop(0, n_chunks, body, 0)
        plsc.subcore_barrier()

        @pl.when(sid == 0)
        def _flush():
            pltpu.sync_copy(acc.at[pl.ds(0, half)], out_hbm.at[cid])

    return k(vals, dst, zeros)


def _segsum(vals, idx, n_seg):
    E, D = vals.shape
    acc_need = (n_seg // _NC + 4104) * D
    if acc_need <= 1260000 and E % (400 * _NS) == 0 and n_seg % (2 * _NC) == 0:
        zeros = jnp.zeros((n_seg // _NC + 4104, D), jnp.float32)
        parts = _sc_scatter_small(vals, idx, n_seg, zeros)
        return parts.reshape(n_seg, D)
    return jax.ops.segment_sum(vals, idx, num_segments=n_seg)


def _conv(p, src, dst, n_seg, x, y):
    n_x = x.shape[0]
    n_e = y.shape[0]
    srcbuf, dstbuf, cup = _row_call(
        _pre_body, n_x, [x],
        [*_p2(p["src_gate"]), *_p2(p["dst_update"]), *_p2(p["dst_gate"]),
         *_p2(p["src_update"])],
        [2 * H, 2 * H, H])
    ga = _gather(srcbuf, src)
    gd = _gather(dstbuf, dst)
    v, ynew = _row_call(
        _mid_body, n_e, [ga, gd, y],
        [*_p2(p["edge_gate"]), *_ln2(p["ln_edges"])],
        [2 * H, H])
    sv = _segsum(v, dst, n_seg)
    (xnew,) = _row_call(
        _post_body, n_x, [sv, cup, x], [*_ln2(p["ln_nodes"])], [H])
    return xnew, ynew


def kernel(atom_features, edge_index, r, lg_edge_index, angle_cos, params):
    src, dst = edge_index[0], edge_index[1]
    lsrc, ldst = lg_edge_index[0], lg_edge_index[1]
    n_nodes = atom_features.shape[0]
    n_edges = r.shape[0]

    pa = params["atom_emb"]
    (x,) = _row_call(
        _emb_body, n_nodes, [atom_features],
        [*_p2(pa["lin"]), *_ln2(pa["ln"])], [H])

    pe = params["edge_emb"]
    (y,) = _row_call(
        _edge_emb_body, n_edges, [r],
        [*_p2(pe[0]["lin"]), *_ln2(pe[0]["ln"]),
         *_p2(pe[1]["lin"]), *_ln2(pe[1]["ln"])], [H])

    pz = params["angle_emb"]
    (z,) = _row_call(
        _angle_emb_body, angle_cos.shape[0], [angle_cos.reshape(-1, 1)],
        [*_p2(pz[0]["lin"]), *_ln2(pz[0]["ln"]),
         *_p2(pz[1]["lin"]), *_ln2(pz[1]["ln"])], [H])

    for layer in params["alignn"]:
        x, m = _conv(layer["node"], src, dst, n_nodes, x, y)
        y, z = _conv(layer["edge"], lsrc, ldst, n_edges, m, z)

    for p in params["gcn"]:
        x, y = _conv(p, src, dst, n_nodes, x, y)

    pf = params["fc"]
    out = pl.pallas_call(
        _final_body,
        in_specs=[pl.BlockSpec((n_nodes, H), lambda: (0, 0)),
                  pl.BlockSpec((H, 1), lambda: (0, 0)),
                  pl.BlockSpec((1, 1), lambda: (0, 0))],
        out_specs=pl.BlockSpec((1, 1), lambda: (0, 0)),
        out_shape=jax.ShapeDtypeStruct((1, 1), jnp.float32),
    )(x, pf["W"], pf["b"].reshape(1, 1))
    return out.reshape(1)


# speedup vs baseline: 1.1517x; 1.0037x over previous
import functools

import jax
import jax.numpy as jnp
from jax import lax
from jax.experimental import pallas as pl
from jax.experimental.pallas import tpu as pltpu
from jax.experimental.pallas import tpu_sc as plsc

H = 64
BLK = 1000
_NC, _NS = 2, 16
_NW = _NC * _NS


def _row_call(body, rows, row_ins, full_ins, out_widths, blk=BLK):
    grid = (rows // blk,)
    in_specs = [pl.BlockSpec((blk, a.shape[1]), lambda i: (i, 0)) for a in row_ins]
    in_specs += [pl.BlockSpec(a.shape, lambda i: (0, 0)) for a in full_ins]
    out_specs = [pl.BlockSpec((blk, w), lambda i: (i, 0)) for w in out_widths]
    out_shape = [jax.ShapeDtypeStruct((rows, w), jnp.float32) for w in out_widths]
    f = pl.pallas_call(
        body,
        grid=grid,
        in_specs=in_specs,
        out_specs=out_specs,
        out_shape=out_shape,
    )
    return f(*row_ins, *full_ins)


def _dot(a, b):
    return jax.lax.dot(a, b, precision=jax.lax.Precision.HIGHEST)


def _ln(v, g, b):
    mu = jnp.mean(v, axis=-1, keepdims=True)
    var = jnp.mean((v - mu) ** 2, axis=-1, keepdims=True)
    return (v - mu) / jnp.sqrt(var + 1e-5) * g + b


def _silu(v):
    return v * jax.nn.sigmoid(v)


def _mlp_blk(v, w, b, g, bb):
    return _silu(_ln(_dot(v, w) + b, g, bb))


def _emb_body(a_ref, w_ref, b_ref, g_ref, bb_ref, o_ref):
    o_ref[...] = _mlp_blk(a_ref[...], w_ref[...], b_ref[...], g_ref[...], bb_ref[...])


def _rbf(x, vmin, vmax, bins):
    step = (vmax - vmin) / (bins - 1)
    idx = jax.lax.broadcasted_iota(jnp.int32, (1, bins), 1).astype(jnp.float32)
    centers = vmin + idx * step
    gamma = 1.0 / step**2
    return jnp.exp(-gamma * (x - centers) ** 2)


def _edge_emb_body(r_ref, w1, b1, g1, bb1, w2, b2, g2, bb2, o_ref):
    r = r_ref[...]
    d = jnp.sqrt(jnp.sum(r * r, axis=1, keepdims=True) + 1e-12)
    y = _rbf(d, 0.0, 8.0, 80)
    y = _mlp_blk(y, w1[...], b1[...], g1[...], bb1[...])
    o_ref[...] = _mlp_blk(y, w2[...], b2[...], g2[...], bb2[...])


def _angle_emb_body(c_ref, w1, b1, g1, bb1, w2, b2, g2, bb2, o_ref):
    z = _rbf(c_ref[...], -1.0, 1.0, 40)
    z = _mlp_blk(z, w1[...], b1[...], g1[...], bb1[...])
    o_ref[...] = _mlp_blk(z, w2[...], b2[...], g2[...], bb2[...])


def _pre_body(x_ref, wa, ba, wb, bb_, wd, bd, wc, bc, src_ref, dst_ref, cup_ref):
    x = x_ref[...]
    a = _dot(x, wa[...]) + ba[...]
    bh = _dot(x, wb[...]) + bb_[...]
    src_ref[...] = jnp.concatenate([a, bh], axis=-1)
    d = _dot(x, wd[...]) + bd[...]
    dst_ref[...] = jnp.concatenate([d, jnp.zeros_like(d)], axis=-1)
    cup_ref[...] = _dot(x, wc[...]) + bc[...]


def _mid_body(ga_ref, gd_ref, y_ref, we, be, g, bb_, v_ref, ynew_ref):
    ga = ga_ref[...]
    y = y_ref[...]
    m = ga[:, :H] + gd_ref[...][:, :H] + (_dot(y, we[...]) + be[...])
    sig = jax.nn.sigmoid(m)
    v_ref[...] = jnp.concatenate([sig * ga[:, H:], sig], axis=-1)
    ynew_ref[...] = y + _silu(_ln(m, g[...], bb_[...]))


def _post_body(s_ref, cup_ref, x_ref, g, bb_, xnew_ref):
    sv = s_ref[...]
    h = sv[:, :H] / (sv[:, H:] + 1e-6)
    xnew_ref[...] = x_ref[...] + _silu(_ln(cup_ref[...] + h, g[...], bb_[...]))


def _final_body(x_ref, w_ref, b_ref, o_ref):
    n = x_ref.shape[0]
    h = jnp.sum(x_ref[...], axis=0, keepdims=True) * (1.0 / n)
    o_ref[...] = (_dot(h, w_ref[...]) + b_ref[...]) * float(n)


def _p2(p):
    return p["W"], p["b"].reshape(1, -1)


def _ln2(p):
    return p["g"].reshape(1, -1), p["b"].reshape(1, -1)


def _sc_gather_call(table, idx, chunk):
    B = idx.shape[0]
    D = table.shape[1]
    per_w = B // _NW
    n_chunks = per_w // chunk
    mesh = plsc.VectorSubcoreMesh(core_axis_name="c", subcore_axis_name="s")

    @functools.partial(
        pl.kernel, mesh=mesh,
        out_type=jax.ShapeDtypeStruct((B, D), jnp.float32),
        scratch_types=[
            pltpu.VMEM((per_w,), jnp.int32),
            pltpu.VMEM((chunk, D), jnp.float32),
            pltpu.VMEM((chunk, D), jnp.float32),
            pltpu.SemaphoreType.DMA,
            pltpu.SemaphoreType.DMA,
        ],
    )
    def k(table_hbm, idx_hbm, out_hbm, idx_v, rows_a, rows_b, sem_a, sem_b):
        wid = lax.axis_index("s") * _NC + lax.axis_index("c")
        base = wid * per_w
        pltpu.sync_copy(idx_hbm.at[pl.ds(base, per_w)], idx_v)
        bufs = (rows_a, rows_b)
        sems = (sem_a, sem_b)
        pend = pltpu.async_copy(
            table_hbm.at[idx_v.at[pl.ds(0, chunk)]], bufs[0], sems[0])
        for j in range(1, n_chunks):
            nxt = pltpu.async_copy(
                table_hbm.at[idx_v.at[pl.ds(j * chunk, chunk)]],
                bufs[j % 2], sems[j % 2])
            pend.wait()
            pltpu.sync_copy(bufs[(j - 1) % 2],
                            out_hbm.at[pl.ds(base + (j - 1) * chunk, chunk)])
            pend = nxt
        pend.wait()
        pltpu.sync_copy(bufs[(n_chunks - 1) % 2],
                        out_hbm.at[pl.ds(base + (n_chunks - 1) * chunk, chunk)])

    return k(table, idx)


def _gather(table, idx):
    B = idx.shape[0]
    chunk = 400 if B % (400 * _NW) == 0 else 328
    unit = chunk * _NW
    Bp = ((B + unit - 1) // unit) * unit
    if Bp != B:
        pad = jnp.arange(Bp - B, dtype=jnp.int32) % table.shape[0]
        idx = jnp.concatenate([idx, pad])
    out = _sc_gather_call(table, idx, chunk)
    return out[:B] if Bp != B else out


def _sc_scatter_small(vals, dst, n_seg, zeros):
    E, D = vals.shape
    half = n_seg // _NC
    n_dump = 4096
    acc_rows = half + n_dump + 8
    per_t = E // _NS
    chunk = 400
    n_chunks = per_t // chunk
    nvec = chunk // 16
    mesh = plsc.VectorSubcoreMesh(core_axis_name="c", subcore_axis_name="s")

    @functools.partial(
        pl.kernel, mesh=mesh,
        out_type=jax.ShapeDtypeStruct((_NC, half, D), jnp.float32),
        scratch_types=[
            pltpu.VMEM((chunk,), jnp.int32),
            pltpu.VMEM((chunk,), jnp.int32),
            pltpu.VMEM((chunk, D), jnp.float32),
            pltpu.VMEM_SHARED((acc_rows, D), jnp.float32),
            pltpu.SemaphoreType.DMA,
        ],
    )
    def k(vals_hbm, dst_hbm, zeros_hbm, out_hbm, idx_v, lidx_v, rows_v, acc,
          sem):
        cid = lax.axis_index("c")
        sid = lax.axis_index("s")
        base = sid * per_t
        lo = cid * half

        @pl.when(sid == 0)
        def _zero():
            pltpu.sync_copy(zeros_hbm, acc)

        plsc.subcore_barrier()

        def body(j, carry):
            off = base + j * chunk
            pltpu.sync_copy(dst_hbm.at[pl.ds(off, chunk)], idx_v)
            pltpu.sync_copy(vals_hbm.at[pl.ds(off, chunk)], rows_v)
            for i in range(nvec):
                iv = idx_v[pl.ds(i * 16, 16)]
                local = iv - lo
                oor = (local < 0) | (local >= half)
                dump = half + (iv & (n_dump - 1))
                lidx_v[pl.ds(i * 16, 16)] = jnp.where(oor, dump, local)
            pltpu.sync_copy(rows_v, acc.at[lidx_v], add=True)
            return carry

        lax.fori_loop(0, n_chunks, body, 0)
        plsc.subcore_barrier()

        @pl.when(sid == 0)
        def _flush():
            pltpu.sync_copy(acc.at[pl.ds(0, half)], out_hbm.at[cid])

    return k(vals, dst, zeros)


def _segsum(vals, idx, n_seg):
    E, D = vals.shape
    acc_need = (n_seg // _NC + 4104) * D
    if acc_need <= 1260000 and E % (400 * _NS) == 0 and n_seg % (2 * _NC) == 0:
        zeros = jnp.zeros((n_seg // _NC + 4104, D), jnp.float32)
        parts = _sc_scatter_small(vals, idx, n_seg, zeros)
        return parts.reshape(n_seg, D)
    return jax.ops.segment_sum(vals, idx, num_segments=n_seg)


def _conv(p, src, dst, n_seg, x, y):
    n_x = x.shape[0]
    n_e = y.shape[0]
    srcbuf, dstbuf, cup = _row_call(
        _pre_body, n_x, [x],
        [*_p2(p["src_gate"]), *_p2(p["dst_update"]), *_p2(p["dst_gate"]),
         *_p2(p["src_update"])],
        [2 * H, 2 * H, H])
    ga = _gather(srcbuf, src)
    gd = _gather(dstbuf, dst)
    v, ynew = _row_call(
        _mid_body, n_e, [ga, gd, y],
        [*_p2(p["edge_gate"]), *_ln2(p["ln_edges"])],
        [2 * H, H])
    sv = _segsum(v, dst, n_seg)
    (xnew,) = _row_call(
        _post_body, n_x, [sv, cup, x], [*_ln2(p["ln_nodes"])], [H])
    return xnew, ynew


def kernel(atom_features, edge_index, r, lg_edge_index, angle_cos, params):
    src, dst = edge_index[0], edge_index[1]
    lsrc, ldst = lg_edge_index[0], lg_edge_index[1]
    n_nodes = atom_features.shape[0]
    n_edges = r.shape[0]

    pa = params["atom_emb"]
    (x,) = _row_call(
        _emb_body, n_nodes, [atom_features],
        [*_p2(pa["lin"]), *_ln2(pa["ln"])], [H])

    pe = params["edge_emb"]
    (y,) = _row_call(
        _edge_emb_body, n_edges, [r],
        [*_p2(pe[0]["lin"]), *_ln2(pe[0]["ln"]),
         *_p2(pe[1]["lin"]), *_ln2(pe[1]["ln"])], [H])

    pz = params["angle_emb"]
    (z,) = _row_call(
        _angle_emb_body, angle_cos.shape[0], [angle_cos.reshape(-1, 1)],
        [*_p2(pz[0]["lin"]), *_ln2(pz[0]["ln"]),
         *_p2(pz[1]["lin"]), *_ln2(pz[1]["ln"])], [H])

    for layer in params["alignn"]:
        x, m = _conv(layer["node"], src, dst, n_nodes, x, y)
        y, z = _conv(layer["edge"], lsrc, ldst, n_edges, m, z)

    for p in params["gcn"]:
        x, y = _conv(p, src, dst, n_nodes, x, y)

    pf = params["fc"]
    out = pl.pallas_call(
        _final_body,
        in_specs=[pl.BlockSpec((n_nodes, H), lambda: (0, 0)),
                  pl.BlockSpec((H, 1), lambda: (0, 0)),
                  pl.BlockSpec((1, 1), lambda: (0, 0))],
        out_specs=pl.BlockSpec((1, 1), lambda: (0, 0)),
        out_shape=jax.ShapeDtypeStruct((1, 1), jnp.float32),
    )(x, pf["W"], pf["b"].reshape(1, 1))
    return out.reshape(1)
